# Initial kernel scaffold; baseline (speedup 1.0000x reference)
import functools

import jax
import jax.numpy as jnp
from jax import lax
from jax.experimental import pallas as pl
from jax.experimental.pallas import tpu as pltpu
from jax.experimental.pallas import tpu_sc as plsc

NC = 2
NS = 16
NW = NC * NS
C = 512

_PREC = lax.Precision.HIGHEST


def _dot(a, b):
    return jnp.dot(a, b, precision=_PREC, preferred_element_type=jnp.float32)


def _silu(x):
    return x * jax.nn.sigmoid(x)


def _ln(x, s, b):
    m = jnp.mean(x, axis=-1, keepdims=True)
    v = jnp.var(x, axis=-1, keepdims=True)
    return (x - m) / jnp.sqrt(v + 1e-5) * s + b


def _smooth_leaky(x, a=0.2):
    return (1.0 + a) / 2.0 * x + (1.0 - a) / 2.0 * x * (2.0 * jax.nn.sigmoid(x) - 1.0)


def _nodeproj_body(x_ref, ws_ref, bs_ref, wd_ref, src_ref, dst_ref):
    x = x_ref[...]
    src_ref[...] = _dot(x, ws_ref[...]) + bs_ref[...]
    dst_ref[...] = _dot(x, wd_ref[...])


def _node_proj(x, w_src, b_src2d, w_dst):
    n, d = x.shape
    return pl.pallas_call(
        _nodeproj_body,
        out_shape=(jax.ShapeDtypeStruct((n, d), jnp.float32),
                   jax.ShapeDtypeStruct((n, d), jnp.float32)),
    )(x, w_src, b_src2d, w_dst)


def _gather_body(e, d, srcidx, dstidx, msrc, mdst, outs, outd, idx_v, rows_v, sem):
    wid = lax.axis_index("s") * NC + lax.axis_index("c")
    total_chunks = e // C
    nbase = total_chunks // NW
    nextra = total_chunks % NW
    count = nbase + jnp.where(wid < nextra, 1, 0)

    def one_table(idx_hbm, table_hbm, out_hbm):
        def body(i, _):
            chunk = wid + i * NW
            base = pl.multiple_of(chunk * C, C)
            pltpu.sync_copy(idx_hbm.at[pl.ds(base, C)], idx_v)
            descs = []
            for j in range(C // 128):
                descs.append(pltpu.async_copy(
                    table_hbm.at[idx_v.at[pl.ds(j * 128, 128)]],
                    rows_v.at[pl.ds(j * 128, 128)], sem))
            for desc in descs:
                desc.wait()
            pltpu.sync_copy(rows_v, out_hbm.at[pl.ds(base, C)])
            return 0
        lax.fori_loop(0, count, body, 0)

    one_table(srcidx, msrc, outs)
    one_table(dstidx, mdst, outd)


def _gather(edge_src, edge_dst, msrc, mdst):
    e = edge_src.shape[0]
    d = msrc.shape[1]
    mesh = plsc.VectorSubcoreMesh(core_axis_name="c", subcore_axis_name="s",
                                  num_cores=NC, num_subcores=NS)
    k = pl.kernel(
        functools.partial(_gather_body, e, d),
        out_type=(jax.ShapeDtypeStruct((e, d), jnp.float32),
                  jax.ShapeDtypeStruct((e, d), jnp.float32)),
        mesh=mesh,
        scratch_types=[
            pltpu.VMEM((C,), jnp.int32),
            pltpu.VMEM((C, d), jnp.float32),
            pltpu.SemaphoreType.DMA,
        ],
    )
    return k(edge_src, edge_dst, msrc, mdst)


def _edge_body(srcg, dstg, escal, eattr,
               w1, l1s, l1b, w2, l2s, l2b, w3, off,
               wa, ba, wact, bact, wint, wv, bv, adot,
               out_w, out_ae):
    h = _silu(_ln(_dot(escal[...], w1[...]), l1s[...], l1b[...]))
    h = _silu(_ln(_dot(h, w2[...]), l2s[...], l2b[...]))
    w = _dot(h, w3[...]) + off[...]
    ea = eattr[...]
    msg = (srcg[...] + dstg[...]) * ea * w
    alpha = _dot(msg, wa[...]) + ba[...]
    t = _smooth_leaky(alpha) * adot[...]
    ii = lax.broadcasted_iota(jnp.int32, (128, 16), 0)
    hh = lax.broadcasted_iota(jnp.int32, (128, 16), 1)
    g16 = (ii // 32 == hh).astype(jnp.float32)
    a16 = _dot(t, g16)
    ae16 = jnp.exp(a16)
    h2 = lax.broadcasted_iota(jnp.int32, (16, 128), 0)
    j2 = lax.broadcasted_iota(jnp.int32, (16, 128), 1)
    h16 = (h2 == j2 // 32).astype(jnp.float32)
    aefull = _dot(ae16, h16)
    val = _silu(_dot(msg, wact[...]) + bact[...]) * ea * wint[...]
    val = _dot(val, wv[...]) + bv[...]
    out_w[...] = val * aefull
    out_ae[...] = ae16


def _edge_stage(srcg, dstg, escal, eattr, params):
    e, d = srcg.shape
    rbf = escal.shape[1]
    hid = params["w2"].shape[0]
    b = 2000
    grid = e // b

    def row_spec(width):
        return pl.BlockSpec((b, width), lambda i: (i, 0))

    def full_spec(shape):
        return pl.BlockSpec(shape, lambda i: tuple(0 for _ in shape))

    in_specs = [
        row_spec(d), row_spec(d), row_spec(rbf), row_spec(1),
        full_spec((rbf, hid)), full_spec((1, hid)), full_spec((1, hid)),
        full_spec((hid, hid)), full_spec((1, hid)), full_spec((1, hid)),
        full_spec((hid, d)), full_spec((1, d)),
        full_spec((d, d)), full_spec((1, d)),
        full_spec((d, d)), full_spec((1, d)), full_spec((1, d)),
        full_spec((d, d)), full_spec((1, d)),
        full_spec((1, d)),
    ]
    out_specs = (row_spec(d), row_spec(16))
    return pl.pallas_call(
        _edge_body,
        grid=(grid,),
        in_specs=in_specs,
        out_specs=out_specs,
        out_shape=(jax.ShapeDtypeStruct((e, d), jnp.float32),
                   jax.ShapeDtypeStruct((e, 16), jnp.float32)),
    )(srcg, dstg, escal, eattr,
      params["w1"], params["l1s"], params["l1b"],
      params["w2"], params["l2s"], params["l2b"],
      params["w3"], params["off"],
      params["wa"], params["ba"],
      params["wact"], params["bact"], params["wint"],
      params["wv"], params["bv"], params["adot"])


def _scatter_body(e, n, d, edst2d, w_hbm, ae_hbm, nump, denp,
                  idx_v, wbuf, aebuf, zw, zae, accw, accae, sem):
    cid = lax.axis_index("c")
    sid = lax.axis_index("s")
    wid = sid * NC + cid
    rows_per_tile = n // NS
    r0 = sid * rows_per_tile

    zvec = jnp.zeros((16,), jnp.float32)
    for r in range(25):
        for k in range(8):
            zw[r, pl.ds(k * 16, 16)] = zvec
    for r in range(125):
        zae[r, pl.ds(0, 16)] = zvec
    for j in range(rows_per_tile // 25):
        pltpu.sync_copy(zw, accw.at[pl.ds(r0 + j * 25, 25)])
    for j in range(rows_per_tile // 125):
        pltpu.sync_copy(zae, accae.at[pl.ds(r0 + j * 125, 125)])
    plsc.subcore_barrier()

    total_chunks = e // C
    nbase = total_chunks // NW
    nextra = total_chunks % NW
    count = nbase + jnp.where(wid < nextra, 1, 0)

    def body(i, _):
        chunk = wid + i * NW
        base = pl.multiple_of(chunk * C, C)
        pltpu.sync_copy(edst2d.at[pl.ds(chunk * (C // 128), C // 128)], idx_v)
        pltpu.sync_copy(w_hbm.at[pl.ds(base, C)], wbuf)
        pltpu.sync_copy(ae_hbm.at[pl.ds(base, C)], aebuf)
        descs = []
        for j in range(C // 128):
            descs.append(pltpu.async_copy(
                wbuf.at[pl.ds(j * 128, 128)], accw.at[idx_v.at[j]], sem, add=True))
            descs.append(pltpu.async_copy(
                aebuf.at[pl.ds(j * 128, 128)], accae.at[idx_v.at[j]], sem, add=True))
        for desc in descs:
            desc.wait()
        return 0
    lax.fori_loop(0, count, body, 0)

    plsc.subcore_barrier()
    pltpu.sync_copy(accw.at[pl.ds(r0, rows_per_tile)],
                    nump.at[cid, pl.ds(r0, rows_per_tile)])
    pltpu.sync_copy(accae.at[pl.ds(r0, rows_per_tile)],
                    denp.at[cid, pl.ds(r0, rows_per_tile)])


def _scatter(edst2d, weighted, ae, n):
    e, d = weighted.shape
    mesh = plsc.VectorSubcoreMesh(core_axis_name="c", subcore_axis_name="s",
                                  num_cores=NC, num_subcores=NS)
    k = pl.kernel(
        functools.partial(_scatter_body, e, n, d),
        out_type=(jax.ShapeDtypeStruct((NC, n, d), jnp.float32),
                  jax.ShapeDtypeStruct((NC, n, 16), jnp.float32)),
        mesh=mesh,
        scratch_types=[
            pltpu.VMEM((C // 128, 128), jnp.int32),
            pltpu.VMEM((C, d), jnp.float32),
            pltpu.VMEM((C, 16), jnp.float32),
            pltpu.VMEM((25, d), jnp.float32),
            pltpu.VMEM((125, 16), jnp.float32),
            pltpu.VMEM_SHARED((n, d), jnp.float32),
            pltpu.VMEM_SHARED((n, 16), jnp.float32),
            pltpu.SemaphoreType.DMA,
        ],
    )
    return k(edst2d, weighted, ae)


def _final_body(nump, denp, wp, bp, out):
    num = nump[0] + nump[1]
    den = denp[0] + denp[1]
    h2 = lax.broadcasted_iota(jnp.int32, (16, 128), 0)
    j2 = lax.broadcasted_iota(jnp.int32, (16, 128), 1)
    h16 = (h2 == j2 // 32).astype(jnp.float32)
    denf = _dot(den, h16)
    p = num / (denf + 1e-16)
    out[...] = _dot(p, wp[...]) + bp[...]


def _final(nump, denp, w_proj, b_proj2d):
    n, d = nump.shape[1], nump.shape[2]
    return pl.pallas_call(
        _final_body,
        out_shape=jax.ShapeDtypeStruct((n, d), jnp.float32),
    )(nump, denp, w_proj, b_proj2d)


def kernel(node_input, node_attr, edge_src, edge_dst, edge_attr, edge_scalars,
           batch, W_src, b_src, W_dst, W1, ln1_s, ln1_b, W2, ln2_s, ln2_b, W3,
           offset, W_alpha, b_alpha, W_act, b_act, w_int, W_val, b_val,
           alpha_dot, W_proj, b_proj):
    n, d = node_input.shape
    e = edge_src.shape[0]
    msrc, mdst = _node_proj(node_input, W_src, b_src.reshape(1, d), W_dst)
    srcg, dstg = _gather(edge_src, edge_dst, msrc, mdst)
    params = {
        "w1": W1, "l1s": ln1_s.reshape(1, -1), "l1b": ln1_b.reshape(1, -1),
        "w2": W2, "l2s": ln2_s.reshape(1, -1), "l2b": ln2_b.reshape(1, -1),
        "w3": W3, "off": offset.reshape(1, d),
        "wa": W_alpha, "ba": b_alpha.reshape(1, d),
        "wact": W_act, "bact": b_act.reshape(1, d),
        "wint": w_int.reshape(1, d),
        "wv": W_val, "bv": b_val.reshape(1, d),
        "adot": alpha_dot.reshape(1, d),
    }
    weighted, ae = _edge_stage(srcg, dstg, edge_scalars, edge_attr, params)
    nump, denp = _scatter(edge_dst.reshape(e // 128, 128), weighted, ae, n)
    return _final(nump, denp, W_proj, b_proj.reshape(1, d))


# speedup vs baseline: 2.0724x; 2.0724x over previous
import functools

import jax
import jax.numpy as jnp
from jax import lax
from jax.experimental import pallas as pl
from jax.experimental.pallas import tpu as pltpu
from jax.experimental.pallas import tpu_sc as plsc

NC = 2
NS = 16
NW = NC * NS
C = 512
CS = 128

_PREC = lax.Precision.HIGHEST


def _dot(a, b):
    return jnp.dot(a, b, precision=_PREC, preferred_element_type=jnp.float32)


def _silu(x):
    return x * jax.nn.sigmoid(x)


def _ln(x, s, b):
    m = jnp.mean(x, axis=-1, keepdims=True)
    v = jnp.var(x, axis=-1, keepdims=True)
    return (x - m) / jnp.sqrt(v + 1e-5) * s + b


def _smooth_leaky(x, a=0.2):
    return (1.0 + a) / 2.0 * x + (1.0 - a) / 2.0 * x * (2.0 * jax.nn.sigmoid(x) - 1.0)


def _nodeproj_body(x_ref, ws_ref, bs_ref, wd_ref, src_ref, dst_ref):
    x = x_ref[...]
    src_ref[...] = _dot(x, ws_ref[...]) + bs_ref[...]
    dst_ref[...] = _dot(x, wd_ref[...])


def _node_proj(x, w_src, b_src2d, w_dst):
    n, d = x.shape
    return pl.pallas_call(
        _nodeproj_body,
        out_shape=(jax.ShapeDtypeStruct((n, d), jnp.float32),
                   jax.ShapeDtypeStruct((n, d), jnp.float32)),
    )(x, w_src, b_src2d, w_dst)


def _gather_body(e, d, srcidx, dstidx, msrc, mdst, outs, outd, idx_v, rows_v, sem):
    wid = lax.axis_index("s") * NC + lax.axis_index("c")
    total_chunks = e // C
    nbase = total_chunks // NW
    nextra = total_chunks % NW
    count = nbase + jnp.where(wid < nextra, 1, 0)

    def one_table(idx_hbm, table_hbm, out_hbm):
        def body(i, _):
            chunk = wid + i * NW
            base = pl.multiple_of(chunk * C, C)
            pltpu.sync_copy(idx_hbm.at[pl.ds(base, C)], idx_v)
            descs = []
            for j in range(C // 128):
                descs.append(pltpu.async_copy(
                    table_hbm.at[idx_v.at[pl.ds(j * 128, 128)]],
                    rows_v.at[pl.ds(j * 128, 128)], sem))
            for desc in descs:
                desc.wait()
            pltpu.sync_copy(rows_v, out_hbm.at[pl.ds(base, C)])
            return 0
        lax.fori_loop(0, count, body, 0)

    one_table(srcidx, msrc, outs)
    one_table(dstidx, mdst, outd)


def _gather(edge_src, edge_dst, msrc, mdst):
    e = edge_src.shape[0]
    d = msrc.shape[1]
    mesh = plsc.VectorSubcoreMesh(core_axis_name="c", subcore_axis_name="s",
                                  num_cores=NC, num_subcores=NS)
    k = pl.kernel(
        functools.partial(_gather_body, e, d),
        out_type=(jax.ShapeDtypeStruct((e, d), jnp.float32),
                  jax.ShapeDtypeStruct((e, d), jnp.float32)),
        mesh=mesh,
        scratch_types=[
            pltpu.VMEM((C,), jnp.int32),
            pltpu.VMEM((C, d), jnp.float32),
            pltpu.SemaphoreType.DMA,
        ],
    )
    return k(edge_src, edge_dst, msrc, mdst)


def _edge_body(srcg, dstg, escal, eattr, edst,
               w1, l1s, l1b, w2, l2s, l2b, w3, off,
               wa, ba, wact, bact, wint, wv, bv, adot,
               out_w, out_ae):
    h = _silu(_ln(_dot(escal[...], w1[...]), l1s[...], l1b[...]))
    h = _silu(_ln(_dot(h, w2[...]), l2s[...], l2b[...]))
    w = _dot(h, w3[...]) + off[...]
    ea = eattr[...]
    msg = (srcg[...] + dstg[...]) * ea * w
    alpha = _dot(msg, wa[...]) + ba[...]
    t = _smooth_leaky(alpha) * adot[...]
    ii = lax.broadcasted_iota(jnp.int32, (128, 16), 0)
    hh = lax.broadcasted_iota(jnp.int32, (128, 16), 1)
    g16 = (ii // 32 == hh).astype(jnp.float32)
    a16 = _dot(t, g16)
    ae16 = jnp.exp(a16)
    h2 = lax.broadcasted_iota(jnp.int32, (16, 128), 0)
    j2 = lax.broadcasted_iota(jnp.int32, (16, 128), 1)
    h16 = (h2 == j2 // 32).astype(jnp.float32)
    aefull = _dot(ae16, h16)
    val = _silu(_dot(msg, wact[...]) + bact[...]) * ea * wint[...]
    val = _dot(val, wv[...]) + bv[...]
    out_w[...] = val * aefull
    tile16 = (j2 % 16 == h2).astype(jnp.float32)
    ae_tiled = _dot(ae16, tile16)
    b = srcg.shape[0]
    m8 = edst[...] & 7
    jj = lax.broadcasted_iota(jnp.int32, (b, 128), 1)
    out_ae[...] = ae_tiled * (jj // 16 == m8).astype(jnp.float32)


def _edge_stage(srcg, dstg, escal, eattr, edstcol, params):
    e, d = srcg.shape
    rbf = escal.shape[1]
    hid = params["w2"].shape[0]
    b = 2000
    grid = e // b

    def row_spec(width):
        return pl.BlockSpec((b, width), lambda i: (i, 0))

    def full_spec(shape):
        return pl.BlockSpec(shape, lambda i: tuple(0 for _ in shape))

    in_specs = [
        row_spec(d), row_spec(d), row_spec(rbf), row_spec(1), row_spec(1),
        full_spec((rbf, hid)), full_spec((1, hid)), full_spec((1, hid)),
        full_spec((hid, hid)), full_spec((1, hid)), full_spec((1, hid)),
        full_spec((hid, d)), full_spec((1, d)),
        full_spec((d, d)), full_spec((1, d)),
        full_spec((d, d)), full_spec((1, d)), full_spec((1, d)),
        full_spec((d, d)), full_spec((1, d)),
        full_spec((1, d)),
    ]
    out_specs = (row_spec(d), row_spec(d))
    return pl.pallas_call(
        _edge_body,
        grid=(grid,),
        in_specs=in_specs,
        out_specs=out_specs,
        out_shape=(jax.ShapeDtypeStruct((e, d), jnp.float32),
                   jax.ShapeDtypeStruct((e, d), jnp.float32)),
    )(srcg, dstg, escal, eattr, edstcol,
      params["w1"], params["l1s"], params["l1b"],
      params["w2"], params["l2s"], params["l2b"],
      params["w3"], params["off"],
      params["wa"], params["ba"],
      params["wact"], params["bact"], params["wint"],
      params["wv"], params["bv"], params["adot"])


def _scatter_body(e, n, d, edst3d, w_hbm, ae_hbm, nump, denp,
                  idx_v, idx8_v, wbuf, aebuf, zw, accw, accae, sem):
    cid = lax.axis_index("c")
    sid = lax.axis_index("s")
    wid = sid * NC + cid
    rows_per_tile = n // NS
    r0 = sid * rows_per_tile
    rows8_per_tile = n // 8 // NS
    r08 = sid * rows8_per_tile

    zvec = jnp.zeros((16,), jnp.float32)
    for r in range(16):
        for k in range(8):
            zw[r, pl.ds(k * 16, 16)] = zvec
    for j in range(rows_per_tile // 16):
        pltpu.sync_copy(zw, accw.at[pl.ds(r0 + j * 16, 16)])
    for j in range(rows8_per_tile // 16):
        pltpu.sync_copy(zw, accae.at[pl.ds(r08 + j * 16, 16)])
    plsc.subcore_barrier()

    total_chunks = e // CS
    nbase = total_chunks // NW
    nextra = total_chunks % NW
    count = nbase + jnp.where(wid < nextra, 1, 0)

    def body(i, _):
        chunk = wid + i * NW
        base = pl.multiple_of(chunk * CS, CS)
        pltpu.sync_copy(edst3d.at[chunk], idx_v)
        pltpu.sync_copy(w_hbm.at[pl.ds(base, CS)], wbuf)
        pltpu.sync_copy(ae_hbm.at[pl.ds(base, CS)], aebuf)
        for j in range(CS // 128):
            for k in range(8):
                v = idx_v[j, pl.ds(k * 16, 16)]
                idx8_v[j, pl.ds(k * 16, 16)] = lax.shift_right_logical(v, 3)
        descs = []
        for j in range(CS // 128):
            descs.append(pltpu.async_copy(
                wbuf.at[pl.ds(j * 128, 128)], accw.at[idx_v.at[j]], sem, add=True))
            descs.append(pltpu.async_copy(
                aebuf.at[pl.ds(j * 128, 128)], accae.at[idx8_v.at[j]], sem, add=True))
        for desc in descs:
            desc.wait()
        return 0
    lax.fori_loop(0, count, body, 0)

    plsc.subcore_barrier()
    pltpu.sync_copy(accw.at[pl.ds(r0, rows_per_tile)],
                    nump.at[cid, pl.ds(r0, rows_per_tile)])
    pltpu.sync_copy(accae.at[pl.ds(r08, rows8_per_tile)],
                    denp.at[cid, pl.ds(r08, rows8_per_tile)])


def _scatter(edst3d, weighted, ae, npad):
    e, d = weighted.shape
    mesh = plsc.VectorSubcoreMesh(core_axis_name="c", subcore_axis_name="s",
                                  num_cores=NC, num_subcores=NS)
    k = pl.kernel(
        functools.partial(_scatter_body, e, npad, d),
        out_type=(jax.ShapeDtypeStruct((NC, npad, d), jnp.float32),
                  jax.ShapeDtypeStruct((NC, npad // 8, d), jnp.float32)),
        mesh=mesh,
        scratch_types=[
            pltpu.VMEM((CS // 128, 128), jnp.int32),
            pltpu.VMEM((CS // 128, 128), jnp.int32),
            pltpu.VMEM((CS, d), jnp.float32),
            pltpu.VMEM((CS, d), jnp.float32),
            pltpu.VMEM((16, d), jnp.float32),
            pltpu.VMEM_SHARED((npad, d), jnp.float32),
            pltpu.VMEM_SHARED((npad // 8, d), jnp.float32),
            pltpu.SemaphoreType.DMA,
        ],
    )
    return k(edst3d, weighted, ae)


def _final_body(nump, denp, wp, bp, out):
    num = nump[0] + nump[1]
    den = denp[0] + denp[1]
    h2 = lax.broadcasted_iota(jnp.int32, (16, 128), 0)
    j2 = lax.broadcasted_iota(jnp.int32, (16, 128), 1)
    h16 = (h2 == j2 // 32).astype(jnp.float32)
    denf = _dot(den, h16)
    p = num / (denf + 1e-16)
    out[...] = _dot(p, wp[...]) + bp[...]


def _final(nump, denp, w_proj, b_proj2d, n):
    d = nump.shape[2]
    b = 2000
    return pl.pallas_call(
        _final_body,
        grid=(n // b,),
        in_specs=[
            pl.BlockSpec((NC, b, d), lambda i: (0, i, 0)),
            pl.BlockSpec((NC, b, 16), lambda i: (0, i, 0)),
            pl.BlockSpec((d, d), lambda i: (0, 0)),
            pl.BlockSpec((1, d), lambda i: (0, 0)),
        ],
        out_specs=pl.BlockSpec((b, d), lambda i: (i, 0)),
        out_shape=jax.ShapeDtypeStruct((n, d), jnp.float32),
    )(nump, denp, w_proj, b_proj2d)


def kernel(node_input, node_attr, edge_src, edge_dst, edge_attr, edge_scalars,
           batch, W_src, b_src, W_dst, W1, ln1_s, ln1_b, W2, ln2_s, ln2_b, W3,
           offset, W_alpha, b_alpha, W_act, b_act, w_int, W_val, b_val,
           alpha_dot, W_proj, b_proj):
    n, d = node_input.shape
    e = edge_src.shape[0]
    msrc, mdst = _node_proj(node_input, W_src, b_src.reshape(1, d), W_dst)
    srcg, dstg = _gather(edge_src, edge_dst, msrc, mdst)
    params = {
        "w1": W1, "l1s": ln1_s.reshape(1, -1), "l1b": ln1_b.reshape(1, -1),
        "w2": W2, "l2s": ln2_s.reshape(1, -1), "l2b": ln2_b.reshape(1, -1),
        "w3": W3, "off": offset.reshape(1, d),
        "wa": W_alpha, "ba": b_alpha.reshape(1, d),
        "wact": W_act, "bact": b_act.reshape(1, d),
        "wint": w_int.reshape(1, d),
        "wv": W_val, "bv": b_val.reshape(1, d),
        "adot": alpha_dot.reshape(1, d),
    }
    weighted, ae = _edge_stage(srcg, dstg, edge_scalars, edge_attr,
                               edge_dst.reshape(e, 1), params)
    npad = ((n + NS * 128 - 1) // (NS * 128)) * NS * 128
    nump, denp = _scatter(edge_dst.reshape(e // CS, CS // 128, 128), weighted,
                          ae, npad)
    den_u = denp.reshape(NC, npad, 16)
    return _final(nump, den_u, W_proj, b_proj.reshape(1, d), n)


# speedup vs baseline: 2.7743x; 1.3387x over previous
import functools

import jax
import jax.numpy as jnp
from jax import lax
from jax.experimental import pallas as pl
from jax.experimental.pallas import tpu as pltpu
from jax.experimental.pallas import tpu_sc as plsc

NC = 2
NS = 16
NW = NC * NS
C = 512
CS = 128

def _dot(a, b, prec=lax.Precision.HIGHEST):
    return jnp.dot(a, b, precision=prec, preferred_element_type=jnp.float32)


def _silu(x):
    return x * jax.nn.sigmoid(x)


def _ln(x, s, b):
    m = jnp.mean(x, axis=-1, keepdims=True)
    v = jnp.var(x, axis=-1, keepdims=True)
    return (x - m) / jnp.sqrt(v + 1e-5) * s + b


def _smooth_leaky(x, a=0.2):
    return (1.0 + a) / 2.0 * x + (1.0 - a) / 2.0 * x * (2.0 * jax.nn.sigmoid(x) - 1.0)


def _nodeproj_body(x_ref, ws_ref, bs_ref, wd_ref, src_ref, dst_ref):
    x = x_ref[...]
    src_ref[...] = _dot(x, ws_ref[...]) + bs_ref[...]
    dst_ref[...] = _dot(x, wd_ref[...])


def _node_proj(x, w_src, b_src2d, w_dst):
    n, d = x.shape
    return pl.pallas_call(
        _nodeproj_body,
        out_shape=(jax.ShapeDtypeStruct((n, d), jnp.float32),
                   jax.ShapeDtypeStruct((n, d), jnp.float32)),
    )(x, w_src, b_src2d, w_dst)


def _gather_body(e, d, srcidx, dstidx, msrc, mdst, outs, outd, idx_v, rows_v, sem):
    wid = lax.axis_index("s") * NC + lax.axis_index("c")
    total_chunks = e // C
    nbase = total_chunks // NW
    nextra = total_chunks % NW
    count = nbase + jnp.where(wid < nextra, 1, 0)

    def one_table(idx_hbm, table_hbm, out_hbm):
        def body(i, _):
            chunk = wid + i * NW
            base = pl.multiple_of(chunk * C, C)
            pltpu.sync_copy(idx_hbm.at[pl.ds(base, C)], idx_v)
            descs = []
            for j in range(C // 128):
                descs.append(pltpu.async_copy(
                    table_hbm.at[idx_v.at[pl.ds(j * 128, 128)]],
                    rows_v.at[pl.ds(j * 128, 128)], sem))
            for desc in descs:
                desc.wait()
            pltpu.sync_copy(rows_v, out_hbm.at[pl.ds(base, C)])
            return 0
        lax.fori_loop(0, count, body, 0)

    one_table(srcidx, msrc, outs)
    one_table(dstidx, mdst, outd)


def _gather(edge_src, edge_dst, msrc, mdst):
    e = edge_src.shape[0]
    d = msrc.shape[1]
    mesh = plsc.VectorSubcoreMesh(core_axis_name="c", subcore_axis_name="s",
                                  num_cores=NC, num_subcores=NS)
    k = pl.kernel(
        functools.partial(_gather_body, e, d),
        out_type=(jax.ShapeDtypeStruct((e, d), jnp.float32),
                  jax.ShapeDtypeStruct((e, d), jnp.float32)),
        mesh=mesh,
        scratch_types=[
            pltpu.VMEM((C,), jnp.int32),
            pltpu.VMEM((C, d), jnp.float32),
            pltpu.SemaphoreType.DMA,
        ],
    )
    return k(edge_src, edge_dst, msrc, mdst)


def _edge_body(srcg, dstg, escal, eattr, edst,
               w1, l1s, l1b, w2, l2s, l2b, w3, off,
               wa, ba, wact, bact, wint, wv, bv, adot,
               g16, h16, tile16, s8,
               out_w, out_ae):
    fast = lax.Precision.DEFAULT
    h = _silu(_ln(_dot(escal[...], w1[...], fast), l1s[...], l1b[...]))
    h = _silu(_ln(_dot(h, w2[...], fast), l2s[...], l2b[...]))
    w = _dot(h, w3[...], fast) + off[...]
    ea = eattr[...]
    msg = (srcg[...] + dstg[...]) * ea * w
    alpha = _dot(msg, wa[...], fast) + ba[...]
    t = _smooth_leaky(alpha) * adot[...]
    a16 = _dot(t, g16[...])
    ae16 = jnp.exp(a16)
    aefull = _dot(ae16, h16[...])
    val = _silu(_dot(msg, wact[...], fast) + bact[...]) * ea * wint[...]
    val = _dot(val, wv[...], fast) + bv[...]
    out_w[...] = val * aefull
    ae_tiled = _dot(ae16, tile16[...])
    b = srcg.shape[0]
    m8 = edst[...] & 7
    i8 = lax.broadcasted_iota(jnp.int32, (b, 8), 1)
    oh8 = (i8 == m8).astype(jnp.float32)
    out_ae[...] = ae_tiled * _dot(oh8, s8[...])


def _edge_stage(srcg, dstg, escal, eattr, edstcol, params):
    e, d = srcg.shape
    rbf = escal.shape[1]
    hid = params["w2"].shape[0]
    b = 2000
    grid = e // b

    def row_spec(width):
        return pl.BlockSpec((b, width), lambda i: (i, 0))

    def full_spec(shape):
        return pl.BlockSpec(shape, lambda i: tuple(0 for _ in shape))

    in_specs = [
        row_spec(d), row_spec(d), row_spec(rbf), row_spec(1), row_spec(1),
        full_spec((rbf, hid)), full_spec((1, hid)), full_spec((1, hid)),
        full_spec((hid, hid)), full_spec((1, hid)), full_spec((1, hid)),
        full_spec((hid, d)), full_spec((1, d)),
        full_spec((d, d)), full_spec((1, d)),
        full_spec((d, d)), full_spec((1, d)), full_spec((1, d)),
        full_spec((d, d)), full_spec((1, d)),
        full_spec((1, d)),
        full_spec((d, 16)), full_spec((16, d)), full_spec((16, d)),
        full_spec((8, d)),
    ]
    out_specs = (row_spec(d), row_spec(d))
    return pl.pallas_call(
        _edge_body,
        grid=(grid,),
        in_specs=in_specs,
        out_specs=out_specs,
        out_shape=(jax.ShapeDtypeStruct((e, d), jnp.float32),
                   jax.ShapeDtypeStruct((e, d), jnp.float32)),
    )(srcg, dstg, escal, eattr, edstcol,
      params["w1"], params["l1s"], params["l1b"],
      params["w2"], params["l2s"], params["l2b"],
      params["w3"], params["off"],
      params["wa"], params["ba"],
      params["wact"], params["bact"], params["wint"],
      params["wv"], params["bv"], params["adot"],
      params["g16"], params["h16"], params["tile16"], params["s8"])


def _scatter_body(e, n, d, edst3d, w_hbm, ae_hbm, nump, denp,
                  idx_v, idx8_v, wbuf, aebuf, zw, accw, accae, sem):
    cid = lax.axis_index("c")
    sid = lax.axis_index("s")
    wid = sid * NC + cid
    rows_per_tile = n // NS
    r0 = sid * rows_per_tile
    rows8_per_tile = n // 8 // NS
    r08 = sid * rows8_per_tile

    zvec = jnp.zeros((16,), jnp.float32)
    for r in range(16):
        for k in range(8):
            zw[r, pl.ds(k * 16, 16)] = zvec
    for j in range(rows_per_tile // 16):
        pltpu.sync_copy(zw, accw.at[pl.ds(r0 + j * 16, 16)])
    for j in range(rows8_per_tile // 16):
        pltpu.sync_copy(zw, accae.at[pl.ds(r08 + j * 16, 16)])
    plsc.subcore_barrier()

    total_chunks = e // CS
    nbase = total_chunks // NW
    nextra = total_chunks % NW
    count = nbase + jnp.where(wid < nextra, 1, 0)

    def body(i, _):
        chunk = wid + i * NW
        base = pl.multiple_of(chunk * CS, CS)
        pltpu.sync_copy(edst3d.at[chunk], idx_v)
        pltpu.sync_copy(w_hbm.at[pl.ds(base, CS)], wbuf)
        pltpu.sync_copy(ae_hbm.at[pl.ds(base, CS)], aebuf)
        for j in range(CS // 128):
            for k in range(8):
                v = idx_v[j, pl.ds(k * 16, 16)]
                idx8_v[j, pl.ds(k * 16, 16)] = lax.shift_right_logical(v, 3)
        descs = []
        for j in range(CS // 128):
            descs.append(pltpu.async_copy(
                wbuf.at[pl.ds(j * 128, 128)], accw.at[idx_v.at[j]], sem, add=True))
            descs.append(pltpu.async_copy(
                aebuf.at[pl.ds(j * 128, 128)], accae.at[idx8_v.at[j]], sem, add=True))
        for desc in descs:
            desc.wait()
        return 0
    lax.fori_loop(0, count, body, 0)

    plsc.subcore_barrier()
    pltpu.sync_copy(accw.at[pl.ds(r0, rows_per_tile)],
                    nump.at[cid, pl.ds(r0, rows_per_tile)])
    pltpu.sync_copy(accae.at[pl.ds(r08, rows8_per_tile)],
                    denp.at[cid, pl.ds(r08, rows8_per_tile)])


def _scatter(edst3d, weighted, ae, npad):
    e, d = weighted.shape
    mesh = plsc.VectorSubcoreMesh(core_axis_name="c", subcore_axis_name="s",
                                  num_cores=NC, num_subcores=NS)
    k = pl.kernel(
        functools.partial(_scatter_body, e, npad, d),
        out_type=(jax.ShapeDtypeStruct((NC, npad, d), jnp.float32),
                  jax.ShapeDtypeStruct((NC, npad // 8, d), jnp.float32)),
        mesh=mesh,
        scratch_types=[
            pltpu.VMEM((CS // 128, 128), jnp.int32),
            pltpu.VMEM((CS // 128, 128), jnp.int32),
            pltpu.VMEM((CS, d), jnp.float32),
            pltpu.VMEM((CS, d), jnp.float32),
            pltpu.VMEM((16, d), jnp.float32),
            pltpu.VMEM_SHARED((npad, d), jnp.float32),
            pltpu.VMEM_SHARED((npad // 8, d), jnp.float32),
            pltpu.SemaphoreType.DMA,
        ],
    )
    return k(edst3d, weighted, ae)


def _final_body(nump, denp, wp, bp, out):
    num = nump[0] + nump[1]
    den = denp[0] + denp[1]
    h2 = lax.broadcasted_iota(jnp.int32, (16, 128), 0)
    j2 = lax.broadcasted_iota(jnp.int32, (16, 128), 1)
    h16 = (h2 == j2 // 32).astype(jnp.float32)
    denf = _dot(den, h16)
    p = num / (denf + 1e-16)
    out[...] = _dot(p, wp[...]) + bp[...]


def _final(nump, denp, w_proj, b_proj2d, n):
    d = nump.shape[2]
    b = 2000
    return pl.pallas_call(
        _final_body,
        grid=(n // b,),
        in_specs=[
            pl.BlockSpec((NC, b, d), lambda i: (0, i, 0)),
            pl.BlockSpec((NC, b, 16), lambda i: (0, i, 0)),
            pl.BlockSpec((d, d), lambda i: (0, 0)),
            pl.BlockSpec((1, d), lambda i: (0, 0)),
        ],
        out_specs=pl.BlockSpec((b, d), lambda i: (i, 0)),
        out_shape=jax.ShapeDtypeStruct((n, d), jnp.float32),
    )(nump, denp, w_proj, b_proj2d)


def kernel(node_input, node_attr, edge_src, edge_dst, edge_attr, edge_scalars,
           batch, W_src, b_src, W_dst, W1, ln1_s, ln1_b, W2, ln2_s, ln2_b, W3,
           offset, W_alpha, b_alpha, W_act, b_act, w_int, W_val, b_val,
           alpha_dot, W_proj, b_proj):
    n, d = node_input.shape
    e = edge_src.shape[0]
    msrc, mdst = _node_proj(node_input, W_src, b_src.reshape(1, d), W_dst)
    srcg, dstg = _gather(edge_src, edge_dst, msrc, mdst)
    params = {
        "w1": W1, "l1s": ln1_s.reshape(1, -1), "l1b": ln1_b.reshape(1, -1),
        "w2": W2, "l2s": ln2_s.reshape(1, -1), "l2b": ln2_b.reshape(1, -1),
        "w3": W3, "off": offset.reshape(1, d),
        "wa": W_alpha, "ba": b_alpha.reshape(1, d),
        "wact": W_act, "bact": b_act.reshape(1, d),
        "wint": w_int.reshape(1, d),
        "wv": W_val, "bv": b_val.reshape(1, d),
        "adot": alpha_dot.reshape(1, d),
    }
    ii = jnp.arange(d)[:, None]
    hh = jnp.arange(16)[None, :]
    params["g16"] = (ii // 32 == hh).astype(jnp.float32)
    jj = jnp.arange(d)[None, :]
    h2 = jnp.arange(16)[:, None]
    params["h16"] = (h2 == jj // 32).astype(jnp.float32)
    params["tile16"] = (jj % 16 == h2).astype(jnp.float32)
    m2 = jnp.arange(8)[:, None]
    params["s8"] = (jj // 16 == m2).astype(jnp.float32)
    weighted, ae = _edge_stage(srcg, dstg, edge_scalars, edge_attr,
                               edge_dst.reshape(e, 1), params)
    npad = ((n + NS * 128 - 1) // (NS * 128)) * NS * 128
    nump, denp = _scatter(edge_dst.reshape(e // CS, CS // 128, 128), weighted,
                          ae, npad)
    den_u = denp.reshape(NC, npad, 16)
    return _final(nump, den_u, W_proj, b_proj.reshape(1, d), n)
